# baseline (speedup 1.0000x reference)
import functools

import jax
import jax.numpy as jnp
from jax import lax
from jax.experimental import pallas as pl
from jax.experimental.pallas import tpu as pltpu
from jax.experimental.pallas import tpu_sc as plsc

N = 10000
D = 128
E = 320000
NLAYERS = 3
NPAD = 10240
NC = 2
NS = 16
NW = NC * NS
CHB = 128
NFULL = (E // CHB) // NW
NTAIL = (E // CHB) % NW
ROWS_T = NPAD // NS
ZROWS = 64

EW = E // NW
CH = 80
NCHUNK = EW // CH

ALPHA = [1.0 / (i + 1) for i in range(NLAYERS + 1)]

_MESH = plsc.VectorSubcoreMesh(
    core_axis_name="c", subcore_axis_name="s", num_cores=NC, num_subcores=NS)
_SC_PARAMS = pltpu.CompilerParams(needs_layout_passes=False)


def _worker_id():
    cid = lax.axis_index("c")
    sid = lax.axis_index("s")
    return cid, sid, sid * NC + cid


@functools.partial(
    pl.kernel,
    out_type=(jax.ShapeDtypeStruct((NPAD,), jnp.float32),
              jax.ShapeDtypeStruct((NPAD,), jnp.float32)),
    mesh=_MESH,
    scratch_types=dict(
        deg_sh=pltpu.VMEM_SHARED((NPAD,), jnp.float32),
        idx0_v=pltpu.VMEM((CH,), jnp.int32),
        idx1_v=pltpu.VMEM((CH,), jnp.int32),
        ones_v=pltpu.VMEM((CH,), jnp.float32),
        zed_v=pltpu.VMEM((ROWS_T,), jnp.float32),
        semi0=pltpu.SemaphoreType.DMA,
        semi1=pltpu.SemaphoreType.DMA,
    ),
)
def _deg_sc(col_hbm, ones_hbm, zeros_hbm, out0, out1,
            deg_sh, idx0_v, idx1_v, ones_v, zed_v, semi0, semi1):
    cid, sid, wid = _worker_id()
    idx = (idx0_v, idx1_v)
    semi = (semi0, semi1)

    def issue_idx(i, b):
        pltpu.async_copy(
            col_hbm.at[pl.ds(wid * EW + i * CH, CH)], idx[b], semi[b])

    def wait_idx(b):
        pltpu.make_async_copy(
            col_hbm.at[pl.ds(0, CH)], idx[b], semi[b]).wait()

    issue_idx(0, 0)
    issue_idx(1, 1)
    pltpu.sync_copy(ones_hbm, ones_v)
    pltpu.sync_copy(zeros_hbm.at[pl.ds(0, ROWS_T)], zed_v)
    pltpu.sync_copy(zed_v, deg_sh.at[pl.ds(sid * ROWS_T, ROWS_T)])
    plsc.subcore_barrier()

    def pair_body(o, carry):
        for b in (0, 1):
            i = 2 * o + b
            wait_idx(b)
            pltpu.sync_copy(ones_v, deg_sh.at[idx[b]], add=True)

            @pl.when(i + 2 < NCHUNK)
            def _():
                issue_idx(i + 2, b)
        return carry

    lax.fori_loop(0, NCHUNK // 2, pair_body, 0)
    wait_idx(0)
    pltpu.sync_copy(ones_v, deg_sh.at[idx0_v], add=True)
    plsc.subcore_barrier()
    sl = pl.ds(sid * ROWS_T, ROWS_T)

    @pl.when(cid == 0)
    def _():
        pltpu.sync_copy(deg_sh.at[sl], out0.at[sl])

    @pl.when(cid == 1)
    def _():
        pltpu.sync_copy(deg_sh.at[sl], out1.at[sl])


@functools.partial(
    pl.kernel,
    out_type=(jax.ShapeDtypeStruct((NPAD, D), jnp.float32),
              jax.ShapeDtypeStruct((NPAD, D), jnp.float32)),
    mesh=_MESH,
    scratch_types=dict(
        acc_sh=pltpu.VMEM_SHARED((NPAD, D), jnp.float32),
        idxr0=pltpu.VMEM((CHB,), jnp.int32),
        idxr1=pltpu.VMEM((CHB,), jnp.int32),
        idxc0=pltpu.VMEM((CHB,), jnp.int32),
        idxc1=pltpu.VMEM((CHB,), jnp.int32),
        rows0=pltpu.VMEM((CHB, D), jnp.float32),
        rows1=pltpu.VMEM((CHB, D), jnp.float32),
        zed_v=pltpu.VMEM((ZROWS, D), jnp.float32),
        semir0=pltpu.SemaphoreType.DMA,
        semir1=pltpu.SemaphoreType.DMA,
        semic0=pltpu.SemaphoreType.DMA,
        semic1=pltpu.SemaphoreType.DMA,
        semg0=pltpu.SemaphoreType.DMA,
        semg1=pltpu.SemaphoreType.DMA,
        sems0=pltpu.SemaphoreType.DMA,
        sems1=pltpu.SemaphoreType.DMA,
        semz=pltpu.SemaphoreType.DMA,
    ),
)
def _layer_sc(y_hbm, row_hbm, col_hbm, zeros_hbm, out0, out1,
              acc_sh, idxr0, idxr1, idxc0, idxc1, rows0, rows1, zed_v,
              semir0, semir1, semic0, semic1, semg0, semg1, sems0, sems1,
              semz):
    cid, sid, wid = _worker_id()
    idxr = (idxr0, idxr1)
    idxc = (idxc0, idxc1)
    rows = (rows0, rows1)
    semir = (semir0, semir1)
    semic = (semic0, semic1)
    semg = (semg0, semg1)
    sems = (sems0, sems1)

    def off(i):
        return (wid + NW * i) * CHB

    def issue_idxr(i, b):
        pltpu.async_copy(row_hbm.at[pl.ds(off(i), CHB)], idxr[b], semir[b])

    def issue_idxc(i, b):
        pltpu.async_copy(col_hbm.at[pl.ds(off(i), CHB)], idxc[b], semic[b])

    def wait_idxr(b):
        pltpu.make_async_copy(
            row_hbm.at[pl.ds(0, CHB)], idxr[b], semir[b]).wait()

    def wait_idxc(b):
        pltpu.make_async_copy(
            col_hbm.at[pl.ds(0, CHB)], idxc[b], semic[b]).wait()

    def issue_gather(b):
        pltpu.async_copy(y_hbm.at[idxr[b]], rows[b], semg[b])

    def wait_gather(b):
        pltpu.make_async_copy(y_hbm.at[idxr[b]], rows[b], semg[b]).wait()

    def issue_scatter(b):
        pltpu.async_copy(rows[b], acc_sh.at[idxc[b]], sems[b], add=True)

    def wait_scatter(b):
        pltpu.make_async_copy(rows[b], acc_sh.at[idxc[b]], sems[b]).wait()

    issue_idxr(0, 0)
    issue_idxr(1, 1)
    issue_idxc(0, 0)
    wait_idxr(0)
    issue_gather(0)

    pltpu.sync_copy(zeros_hbm.at[pl.ds(0, ZROWS)], zed_v)
    for t in range(ROWS_T // ZROWS):
        pltpu.async_copy(
            zed_v, acc_sh.at[pl.ds(sid * ROWS_T + t * ZROWS, ZROWS)], semz)
    for t in range(ROWS_T // ZROWS):
        pltpu.make_async_copy(
            zed_v, acc_sh.at[pl.ds(sid * ROWS_T, ZROWS)], semz).wait()
    plsc.subcore_barrier()

    def pair_body(o, carry):
        for b in (0, 1):
            j = 2 * o + b
            bp = 1 - b

            @pl.when(j >= 1)
            def _():
                wait_scatter(bp)

            @pl.when(j + 1 < NFULL)
            def _():
                issue_idxc(j + 1, bp)
                wait_idxr(bp)
                issue_gather(bp)

            wait_gather(b)

            @pl.when(j + 2 < NFULL)
            def _():
                issue_idxr(j + 2, b)

            wait_idxc(b)
            issue_scatter(b)
        return carry

    lax.fori_loop(0, NFULL // 2, pair_body, 0)
    wait_scatter((NFULL - 1) % 2)

    @pl.when(wid < NTAIL)
    def _():
        toff = (NW * NFULL + wid) * CHB
        pltpu.sync_copy(row_hbm.at[pl.ds(toff, CHB)], idxr0)
        pltpu.sync_copy(col_hbm.at[pl.ds(toff, CHB)], idxc0)
        pltpu.async_copy(y_hbm.at[idxr0], rows0, semg0).wait()
        pltpu.sync_copy(rows0, acc_sh.at[idxc0], add=True)

    plsc.subcore_barrier()
    sl = pl.ds(sid * ROWS_T, ROWS_T)

    @pl.when(cid == 0)
    def _():
        pltpu.sync_copy(acc_sh.at[sl], out0.at[sl])

    @pl.when(cid == 1)
    def _():
        pltpu.sync_copy(acc_sh.at[sl], out1.at[sl])


def _dot_chunk(a_buf, b_buf, tbuf, res_v):
    lanes = lax.iota(jnp.int32, 16)

    def gbody(g, carry):
        for h in (0, 1):
            base = g * 32 + h * 16
            toff = h * 256
            for e in range(16):
                eg = base + e
                acc = a_buf[eg, pl.ds(0, 16)] * b_buf[eg, pl.ds(0, 16)]
                for j in range(1, D // 16):
                    acc = acc + (a_buf[eg, pl.ds(j * 16, 16)] *
                                 b_buf[eg, pl.ds(j * 16, 16)])
                tbuf[pl.ds(toff + e * 16, 16)] = acc
            tot = plsc.load_gather(tbuf, [lanes * 16 + toff])
            for l in range(1, 16):
                tot = tot + plsc.load_gather(tbuf, [lanes * 16 + toff + l])
            res_v[pl.ds(base, 16)] = tot
        return carry

    lax.fori_loop(0, CHB // 32, gbody, 0)


@functools.partial(
    pl.kernel,
    out_type=jax.ShapeDtypeStruct((E,), jnp.float32),
    mesh=_MESH,
    compiler_params=_SC_PARAMS,
    scratch_types=dict(
        idxr0=pltpu.VMEM((CHB,), jnp.int32),
        idxr1=pltpu.VMEM((CHB,), jnp.int32),
        idxr2=pltpu.VMEM((CHB,), jnp.int32),
        idxc0=pltpu.VMEM((CHB,), jnp.int32),
        idxc1=pltpu.VMEM((CHB,), jnp.int32),
        idxc2=pltpu.VMEM((CHB,), jnp.int32),
        a0=pltpu.VMEM((CHB, D), jnp.float32),
        a1=pltpu.VMEM((CHB, D), jnp.float32),
        a2=pltpu.VMEM((CHB, D), jnp.float32),
        b0=pltpu.VMEM((CHB, D), jnp.float32),
        b1=pltpu.VMEM((CHB, D), jnp.float32),
        b2=pltpu.VMEM((CHB, D), jnp.float32),
        tbuf=pltpu.VMEM((512,), jnp.float32),
        res0=pltpu.VMEM((CHB,), jnp.float32),
        res1=pltpu.VMEM((CHB,), jnp.float32),
        res2=pltpu.VMEM((CHB,), jnp.float32),
        semi0=pltpu.SemaphoreType.DMA,
        semi1=pltpu.SemaphoreType.DMA,
        semi2=pltpu.SemaphoreType.DMA,
        semg0=pltpu.SemaphoreType.DMA,
        semg1=pltpu.SemaphoreType.DMA,
        semg2=pltpu.SemaphoreType.DMA,
        semr0=pltpu.SemaphoreType.DMA,
        semr1=pltpu.SemaphoreType.DMA,
        semr2=pltpu.SemaphoreType.DMA,
    ),
)
def _score_sc(out_hbm, row_hbm, col_hbm, score_hbm,
              idxr0, idxr1, idxr2, idxc0, idxc1, idxc2,
              a0, a1, a2, b0, b1, b2, tbuf, res0, res1, res2,
              semi0, semi1, semi2, semg0, semg1, semg2,
              semr0, semr1, semr2):
    cid, sid, wid = _worker_id()
    idxr = (idxr0, idxr1, idxr2)
    idxc = (idxc0, idxc1, idxc2)
    abuf = (a0, a1, a2)
    bbuf = (b0, b1, b2)
    res = (res0, res1, res2)
    semi = (semi0, semi1, semi2)
    semg = (semg0, semg1, semg2)
    semr = (semr0, semr1, semr2)

    def off(i):
        return (wid + NW * i) * CHB

    def issue_idx(i, b):
        pltpu.async_copy(row_hbm.at[pl.ds(off(i), CHB)], idxr[b], semi[b])
        pltpu.async_copy(col_hbm.at[pl.ds(off(i), CHB)], idxc[b], semi[b])

    def wait_idx(b):
        pltpu.make_async_copy(
            row_hbm.at[pl.ds(0, CHB)], idxr[b], semi[b]).wait()
        pltpu.make_async_copy(
            col_hbm.at[pl.ds(0, CHB)], idxc[b], semi[b]).wait()

    def issue_gathers(b):
        pltpu.async_copy(out_hbm.at[idxr[b]], abuf[b], semg[b])
        pltpu.async_copy(out_hbm.at[idxc[b]], bbuf[b], semg[b])

    def wait_gathers(b):
        pltpu.make_async_copy(out_hbm.at[idxr[b]], abuf[b], semg[b]).wait()
        pltpu.make_async_copy(out_hbm.at[idxc[b]], bbuf[b], semg[b]).wait()

    issue_idx(0, 0)
    issue_idx(1, 1)
    issue_idx(2, 2)
    wait_idx(0)
    issue_gathers(0)
    wait_idx(1)
    issue_gathers(1)

    def trip_body(o, carry):
        for b in (0, 1, 2):
            i = 3 * o + b
            b2 = (b + 2) % 3

            wait_gathers(b)

            @pl.when(i + 2 < NFULL)
            def _():
                wait_idx(b2)
                issue_gathers(b2)

            @pl.when(i + 3 < NFULL)
            def _():
                issue_idx(i + 3, b)

            @pl.when(i >= 3)
            def _():
                pltpu.make_async_copy(
                    res[b], score_hbm.at[pl.ds(0, CHB)], semr[b]).wait()

            _dot_chunk(abuf[b], bbuf[b], tbuf, res[b])
            pltpu.async_copy(res[b], score_hbm.at[pl.ds(off(i), CHB)], semr[b])
        return carry

    lax.fori_loop(0, NFULL // 3, trip_body, 0)
    for b in (0, 1, 2):
        pltpu.make_async_copy(
            res[b], score_hbm.at[pl.ds(0, CHB)], semr[b]).wait()

    @pl.when(wid < NTAIL)
    def _():
        toff = (NW * NFULL + wid) * CHB
        pltpu.sync_copy(row_hbm.at[pl.ds(toff, CHB)], idxr0)
        pltpu.sync_copy(col_hbm.at[pl.ds(toff, CHB)], idxc0)
        pltpu.async_copy(out_hbm.at[idxr0], a0, semg0).wait()
        pltpu.async_copy(out_hbm.at[idxc0], b0, semg0).wait()
        _dot_chunk(a0, b0, tbuf, res0)
        pltpu.sync_copy(res0, score_hbm.at[pl.ds(toff, CHB)])


_BLK = 1024
_GRID = NPAD // _BLK


def _prep_body(d0_ref, d1_ref, emb_ref, dis_ref, y_ref, out_ref):
    deg = d0_ref[...] + d1_ref[...]
    dis = jnp.where(deg > 0, lax.rsqrt(jnp.maximum(deg, 1e-12)), 0.0)
    dis_ref[...] = dis
    y_ref[...] = emb_ref[...] * dis[:, None]
    out_ref[...] = emb_ref[...] * ALPHA[0]


def _tc_prep(d0, d1, emb_p):
    return pl.pallas_call(
        _prep_body,
        grid=(_GRID,),
        in_specs=[
            pl.BlockSpec((_BLK,), lambda i: (i,)),
            pl.BlockSpec((_BLK,), lambda i: (i,)),
            pl.BlockSpec((_BLK, D), lambda i: (i, 0)),
        ],
        out_specs=[
            pl.BlockSpec((_BLK,), lambda i: (i,)),
            pl.BlockSpec((_BLK, D), lambda i: (i, 0)),
            pl.BlockSpec((_BLK, D), lambda i: (i, 0)),
        ],
        out_shape=[
            jax.ShapeDtypeStruct((NPAD,), jnp.float32),
            jax.ShapeDtypeStruct((NPAD, D), jnp.float32),
            jax.ShapeDtypeStruct((NPAD, D), jnp.float32),
        ],
    )(d0, d1, emb_p)


def _comb_body(alpha, p0_ref, p1_ref, dis_ref, prev_ref, y_ref, out_ref):
    dis = dis_ref[...][:, None]
    x = (p0_ref[...] + p1_ref[...]) * dis
    out_ref[...] = prev_ref[...] + alpha * x
    y_ref[...] = x * dis


def _tc_comb(p0, p1, dis, prev, alpha):
    return pl.pallas_call(
        functools.partial(_comb_body, alpha),
        grid=(_GRID,),
        in_specs=[
            pl.BlockSpec((_BLK, D), lambda i: (i, 0)),
            pl.BlockSpec((_BLK, D), lambda i: (i, 0)),
            pl.BlockSpec((_BLK,), lambda i: (i,)),
            pl.BlockSpec((_BLK, D), lambda i: (i, 0)),
        ],
        out_specs=[
            pl.BlockSpec((_BLK, D), lambda i: (i, 0)),
            pl.BlockSpec((_BLK, D), lambda i: (i, 0)),
        ],
        out_shape=[
            jax.ShapeDtypeStruct((NPAD, D), jnp.float32),
            jax.ShapeDtypeStruct((NPAD, D), jnp.float32),
        ],
    )(p0, p1, dis, prev)


def kernel(edge_index, emb_weight):
    row = edge_index[0]
    col = edge_index[1]
    emb_p = jnp.zeros((NPAD, D), jnp.float32).at[:N].set(emb_weight)
    ones_ch = jnp.ones((CH,), jnp.float32)
    zeros1d = jnp.zeros((ROWS_T,), jnp.float32)
    zeros2d = jnp.zeros((ZROWS, D), jnp.float32)

    d0, d1 = _deg_sc(col, ones_ch, zeros1d)
    dis, y, out = _tc_prep(d0, d1, emb_p)
    for k in range(1, NLAYERS + 1):
        p0, p1 = _layer_sc(y, row, col, zeros2d)
        y, out = _tc_comb(p0, p1, dis, out, ALPHA[k])
    return _score_sc(out, row, col)


# speedup vs baseline: 1.0378x; 1.0378x over previous
import functools

import jax
import jax.numpy as jnp
from jax import lax
from jax.experimental import pallas as pl
from jax.experimental.pallas import tpu as pltpu
from jax.experimental.pallas import tpu_sc as plsc

N = 10000
D = 128
E = 320000
NLAYERS = 3
NPAD = 10240
NC = 2
NS = 16
NW = NC * NS
CHB = 128
NFULL = (E // CHB) // NW
NTAIL = (E // CHB) % NW
ROWS_T = NPAD // NS
ZROWS = 64

EW = E // NW
CH = 80
NCHUNK = EW // CH

ALPHA = [1.0 / (i + 1) for i in range(NLAYERS + 1)]

_MESH = plsc.VectorSubcoreMesh(
    core_axis_name="c", subcore_axis_name="s", num_cores=NC, num_subcores=NS)
_SC_PARAMS = pltpu.CompilerParams(needs_layout_passes=False)


def _worker_id():
    cid = lax.axis_index("c")
    sid = lax.axis_index("s")
    return cid, sid, sid * NC + cid


@functools.partial(
    pl.kernel,
    out_type=(jax.ShapeDtypeStruct((NPAD,), jnp.float32),
              jax.ShapeDtypeStruct((NPAD,), jnp.float32)),
    mesh=_MESH,
    scratch_types=dict(
        deg_sh=pltpu.VMEM_SHARED((NPAD,), jnp.float32),
        idx0_v=pltpu.VMEM((CH,), jnp.int32),
        idx1_v=pltpu.VMEM((CH,), jnp.int32),
        ones_v=pltpu.VMEM((CH,), jnp.float32),
        zed_v=pltpu.VMEM((ROWS_T,), jnp.float32),
        semi0=pltpu.SemaphoreType.DMA,
        semi1=pltpu.SemaphoreType.DMA,
    ),
)
def _deg_sc(col_hbm, ones_hbm, zeros_hbm, out0, out1,
            deg_sh, idx0_v, idx1_v, ones_v, zed_v, semi0, semi1):
    cid, sid, wid = _worker_id()
    idx = (idx0_v, idx1_v)
    semi = (semi0, semi1)

    def issue_idx(i, b):
        pltpu.async_copy(
            col_hbm.at[pl.ds(wid * EW + i * CH, CH)], idx[b], semi[b])

    def wait_idx(b):
        pltpu.make_async_copy(
            col_hbm.at[pl.ds(0, CH)], idx[b], semi[b]).wait()

    issue_idx(0, 0)
    issue_idx(1, 1)
    pltpu.sync_copy(ones_hbm, ones_v)
    pltpu.sync_copy(zeros_hbm.at[pl.ds(0, ROWS_T)], zed_v)
    pltpu.sync_copy(zed_v, deg_sh.at[pl.ds(sid * ROWS_T, ROWS_T)])
    plsc.subcore_barrier()

    def pair_body(o, carry):
        for b in (0, 1):
            i = 2 * o + b
            wait_idx(b)
            pltpu.sync_copy(ones_v, deg_sh.at[idx[b]], add=True)

            @pl.when(i + 2 < NCHUNK)
            def _():
                issue_idx(i + 2, b)
        return carry

    lax.fori_loop(0, NCHUNK // 2, pair_body, 0)
    wait_idx(0)
    pltpu.sync_copy(ones_v, deg_sh.at[idx0_v], add=True)
    plsc.subcore_barrier()
    sl = pl.ds(sid * ROWS_T, ROWS_T)

    @pl.when(cid == 0)
    def _():
        pltpu.sync_copy(deg_sh.at[sl], out0.at[sl])

    @pl.when(cid == 1)
    def _():
        pltpu.sync_copy(deg_sh.at[sl], out1.at[sl])


@functools.partial(
    pl.kernel,
    out_type=(jax.ShapeDtypeStruct((NPAD, D), jnp.float32),
              jax.ShapeDtypeStruct((NPAD, D), jnp.float32)),
    mesh=_MESH,
    scratch_types=dict(
        acc_sh=pltpu.VMEM_SHARED((NPAD, D), jnp.float32),
        idxr0=pltpu.VMEM((CHB,), jnp.int32),
        idxr1=pltpu.VMEM((CHB,), jnp.int32),
        idxc0=pltpu.VMEM((CHB,), jnp.int32),
        idxc1=pltpu.VMEM((CHB,), jnp.int32),
        rows0=pltpu.VMEM((CHB, D), jnp.float32),
        rows1=pltpu.VMEM((CHB, D), jnp.float32),
        zed_v=pltpu.VMEM((ZROWS, D), jnp.float32),
        semir0=pltpu.SemaphoreType.DMA,
        semir1=pltpu.SemaphoreType.DMA,
        semic0=pltpu.SemaphoreType.DMA,
        semic1=pltpu.SemaphoreType.DMA,
        semg0=pltpu.SemaphoreType.DMA,
        semg1=pltpu.SemaphoreType.DMA,
        sems0=pltpu.SemaphoreType.DMA,
        sems1=pltpu.SemaphoreType.DMA,
        semz=pltpu.SemaphoreType.DMA,
    ),
)
def _layer_sc(y_hbm, row_hbm, col_hbm, zeros_hbm, out0, out1,
              acc_sh, idxr0, idxr1, idxc0, idxc1, rows0, rows1, zed_v,
              semir0, semir1, semic0, semic1, semg0, semg1, sems0, sems1,
              semz):
    cid, sid, wid = _worker_id()
    idxr = (idxr0, idxr1)
    idxc = (idxc0, idxc1)
    rows = (rows0, rows1)
    semir = (semir0, semir1)
    semic = (semic0, semic1)
    semg = (semg0, semg1)
    sems = (sems0, sems1)

    def off(i):
        return (wid + NW * i) * CHB

    def issue_idxr(i, b):
        pltpu.async_copy(row_hbm.at[pl.ds(off(i), CHB)], idxr[b], semir[b])

    def issue_idxc(i, b):
        pltpu.async_copy(col_hbm.at[pl.ds(off(i), CHB)], idxc[b], semic[b])

    def wait_idxr(b):
        pltpu.make_async_copy(
            row_hbm.at[pl.ds(0, CHB)], idxr[b], semir[b]).wait()

    def wait_idxc(b):
        pltpu.make_async_copy(
            col_hbm.at[pl.ds(0, CHB)], idxc[b], semic[b]).wait()

    def issue_gather(b):
        pltpu.async_copy(y_hbm.at[idxr[b]], rows[b], semg[b])

    def wait_gather(b):
        pltpu.make_async_copy(y_hbm.at[idxr[b]], rows[b], semg[b]).wait()

    def issue_scatter(b):
        pltpu.async_copy(rows[b], acc_sh.at[idxc[b]], sems[b], add=True)

    def wait_scatter(b):
        pltpu.make_async_copy(rows[b], acc_sh.at[idxc[b]], sems[b]).wait()

    issue_idxr(0, 0)
    issue_idxr(1, 1)
    issue_idxc(0, 0)
    wait_idxr(0)
    issue_gather(0)

    pltpu.sync_copy(zeros_hbm.at[pl.ds(0, ZROWS)], zed_v)
    for t in range(ROWS_T // ZROWS):
        pltpu.async_copy(
            zed_v, acc_sh.at[pl.ds(sid * ROWS_T + t * ZROWS, ZROWS)], semz)
    for t in range(ROWS_T // ZROWS):
        pltpu.make_async_copy(
            zed_v, acc_sh.at[pl.ds(sid * ROWS_T, ZROWS)], semz).wait()
    plsc.subcore_barrier()

    def pair_body(o, carry):
        for b in (0, 1):
            j = 2 * o + b
            bp = 1 - b

            @pl.when(j >= 1)
            def _():
                wait_scatter(bp)

            @pl.when(j + 1 < NFULL)
            def _():
                issue_idxc(j + 1, bp)
                wait_idxr(bp)
                issue_gather(bp)

            wait_gather(b)

            @pl.when(j + 2 < NFULL)
            def _():
                issue_idxr(j + 2, b)

            wait_idxc(b)
            issue_scatter(b)
        return carry

    lax.fori_loop(0, NFULL // 2, pair_body, 0)
    wait_scatter((NFULL - 1) % 2)

    @pl.when(wid < NTAIL)
    def _():
        toff = (NW * NFULL + wid) * CHB
        pltpu.sync_copy(row_hbm.at[pl.ds(toff, CHB)], idxr0)
        pltpu.sync_copy(col_hbm.at[pl.ds(toff, CHB)], idxc0)
        pltpu.async_copy(y_hbm.at[idxr0], rows0, semg0).wait()
        pltpu.sync_copy(rows0, acc_sh.at[idxc0], add=True)

    plsc.subcore_barrier()
    sl = pl.ds(sid * ROWS_T, ROWS_T)

    @pl.when(cid == 0)
    def _():
        pltpu.sync_copy(acc_sh.at[sl], out0.at[sl])

    @pl.when(cid == 1)
    def _():
        pltpu.sync_copy(acc_sh.at[sl], out1.at[sl])


def _dot_chunk(a_buf, b_buf, tbuf, res_v):
    lanes = lax.iota(jnp.int32, 16)

    def gbody(g, carry):
        base = g * 16
        for e in range(16):
            eg = base + e
            acc = a_buf[eg, pl.ds(0, 16)] * b_buf[eg, pl.ds(0, 16)]
            for j in range(1, D // 16):
                acc = acc + (a_buf[eg, pl.ds(j * 16, 16)] *
                             b_buf[eg, pl.ds(j * 16, 16)])
            tbuf[pl.ds(e * 16, 16)] = acc
        tot = plsc.load_gather(tbuf, [lanes * 16])
        for l in range(1, 16):
            tot = tot + plsc.load_gather(tbuf, [lanes * 16 + l])
        res_v[pl.ds(base, 16)] = tot
        return carry

    lax.fori_loop(0, CHB // 16, gbody, 0)


@functools.partial(
    pl.kernel,
    out_type=jax.ShapeDtypeStruct((E,), jnp.float32),
    mesh=_MESH,
    compiler_params=_SC_PARAMS,
    scratch_types=dict(
        idxr0=pltpu.VMEM((CHB,), jnp.int32),
        idxr1=pltpu.VMEM((CHB,), jnp.int32),
        idxr2=pltpu.VMEM((CHB,), jnp.int32),
        idxc0=pltpu.VMEM((CHB,), jnp.int32),
        idxc1=pltpu.VMEM((CHB,), jnp.int32),
        idxc2=pltpu.VMEM((CHB,), jnp.int32),
        a0=pltpu.VMEM((CHB, D), jnp.float32),
        a1=pltpu.VMEM((CHB, D), jnp.float32),
        a2=pltpu.VMEM((CHB, D), jnp.float32),
        b0=pltpu.VMEM((CHB, D), jnp.float32),
        b1=pltpu.VMEM((CHB, D), jnp.float32),
        b2=pltpu.VMEM((CHB, D), jnp.float32),
        tbuf=pltpu.VMEM((256,), jnp.float32),
        res0=pltpu.VMEM((CHB,), jnp.float32),
        res1=pltpu.VMEM((CHB,), jnp.float32),
        res2=pltpu.VMEM((CHB,), jnp.float32),
        semi0=pltpu.SemaphoreType.DMA,
        semi1=pltpu.SemaphoreType.DMA,
        semi2=pltpu.SemaphoreType.DMA,
        semg0=pltpu.SemaphoreType.DMA,
        semg1=pltpu.SemaphoreType.DMA,
        semg2=pltpu.SemaphoreType.DMA,
        semr0=pltpu.SemaphoreType.DMA,
        semr1=pltpu.SemaphoreType.DMA,
        semr2=pltpu.SemaphoreType.DMA,
    ),
)
def _score_sc(out_hbm, row_hbm, col_hbm, score_hbm,
              idxr0, idxr1, idxr2, idxc0, idxc1, idxc2,
              a0, a1, a2, b0, b1, b2, tbuf, res0, res1, res2,
              semi0, semi1, semi2, semg0, semg1, semg2,
              semr0, semr1, semr2):
    cid, sid, wid = _worker_id()
    idxr = (idxr0, idxr1, idxr2)
    idxc = (idxc0, idxc1, idxc2)
    abuf = (a0, a1, a2)
    bbuf = (b0, b1, b2)
    res = (res0, res1, res2)
    semi = (semi0, semi1, semi2)
    semg = (semg0, semg1, semg2)
    semr = (semr0, semr1, semr2)

    def off(i):
        return (wid + NW * i) * CHB

    def issue_idx(i, b):
        pltpu.async_copy(row_hbm.at[pl.ds(off(i), CHB)], idxr[b], semi[b])
        pltpu.async_copy(col_hbm.at[pl.ds(off(i), CHB)], idxc[b], semi[b])

    def wait_idx(b):
        pltpu.make_async_copy(
            row_hbm.at[pl.ds(0, CHB)], idxr[b], semi[b]).wait()
        pltpu.make_async_copy(
            col_hbm.at[pl.ds(0, CHB)], idxc[b], semi[b]).wait()

    def issue_gathers(b):
        pltpu.async_copy(out_hbm.at[idxr[b]], abuf[b], semg[b])
        pltpu.async_copy(out_hbm.at[idxc[b]], bbuf[b], semg[b])

    def wait_gathers(b):
        pltpu.make_async_copy(out_hbm.at[idxr[b]], abuf[b], semg[b]).wait()
        pltpu.make_async_copy(out_hbm.at[idxc[b]], bbuf[b], semg[b]).wait()

    issue_idx(0, 0)
    issue_idx(1, 1)
    issue_idx(2, 2)
    wait_idx(0)
    issue_gathers(0)
    wait_idx(1)
    issue_gathers(1)

    def trip_body(o, carry):
        for b in (0, 1, 2):
            i = 3 * o + b
            b2 = (b + 2) % 3

            wait_gathers(b)

            @pl.when(i + 2 < NFULL)
            def _():
                wait_idx(b2)
                issue_gathers(b2)

            @pl.when(i + 3 < NFULL)
            def _():
                issue_idx(i + 3, b)

            @pl.when(i >= 3)
            def _():
                pltpu.make_async_copy(
                    res[b], score_hbm.at[pl.ds(0, CHB)], semr[b]).wait()

            _dot_chunk(abuf[b], bbuf[b], tbuf, res[b])
            pltpu.async_copy(res[b], score_hbm.at[pl.ds(off(i), CHB)], semr[b])
        return carry

    lax.fori_loop(0, NFULL // 3, trip_body, 0)
    for b in (0, 1, 2):
        pltpu.make_async_copy(
            res[b], score_hbm.at[pl.ds(0, CHB)], semr[b]).wait()

    @pl.when(wid < NTAIL)
    def _():
        toff = (NW * NFULL + wid) * CHB
        pltpu.sync_copy(row_hbm.at[pl.ds(toff, CHB)], idxr0)
        pltpu.sync_copy(col_hbm.at[pl.ds(toff, CHB)], idxc0)
        pltpu.async_copy(out_hbm.at[idxr0], a0, semg0).wait()
        pltpu.async_copy(out_hbm.at[idxc0], b0, semg0).wait()
        _dot_chunk(a0, b0, tbuf, res0)
        pltpu.sync_copy(res0, score_hbm.at[pl.ds(toff, CHB)])


_BLK = 1024
_GRID = NPAD // _BLK


def _prep_body(d0_ref, d1_ref, emb_ref, dis_ref, y_ref, out_ref):
    deg = d0_ref[...] + d1_ref[...]
    dis = jnp.where(deg > 0, lax.rsqrt(jnp.maximum(deg, 1e-12)), 0.0)
    dis_ref[...] = dis
    y_ref[...] = emb_ref[...] * dis[:, None]
    out_ref[...] = emb_ref[...] * ALPHA[0]


def _tc_prep(d0, d1, emb_p):
    return pl.pallas_call(
        _prep_body,
        grid=(_GRID,),
        in_specs=[
            pl.BlockSpec((_BLK,), lambda i: (i,)),
            pl.BlockSpec((_BLK,), lambda i: (i,)),
            pl.BlockSpec((_BLK, D), lambda i: (i, 0)),
        ],
        out_specs=[
            pl.BlockSpec((_BLK,), lambda i: (i,)),
            pl.BlockSpec((_BLK, D), lambda i: (i, 0)),
            pl.BlockSpec((_BLK, D), lambda i: (i, 0)),
        ],
        out_shape=[
            jax.ShapeDtypeStruct((NPAD,), jnp.float32),
            jax.ShapeDtypeStruct((NPAD, D), jnp.float32),
            jax.ShapeDtypeStruct((NPAD, D), jnp.float32),
        ],
    )(d0, d1, emb_p)


def _comb_body(alpha, p0_ref, p1_ref, dis_ref, prev_ref, y_ref, out_ref):
    dis = dis_ref[...][:, None]
    x = (p0_ref[...] + p1_ref[...]) * dis
    out_ref[...] = prev_ref[...] + alpha * x
    y_ref[...] = x * dis


def _tc_comb(p0, p1, dis, prev, alpha):
    return pl.pallas_call(
        functools.partial(_comb_body, alpha),
        grid=(_GRID,),
        in_specs=[
            pl.BlockSpec((_BLK, D), lambda i: (i, 0)),
            pl.BlockSpec((_BLK, D), lambda i: (i, 0)),
            pl.BlockSpec((_BLK,), lambda i: (i,)),
            pl.BlockSpec((_BLK, D), lambda i: (i, 0)),
        ],
        out_specs=[
            pl.BlockSpec((_BLK, D), lambda i: (i, 0)),
            pl.BlockSpec((_BLK, D), lambda i: (i, 0)),
        ],
        out_shape=[
            jax.ShapeDtypeStruct((NPAD, D), jnp.float32),
            jax.ShapeDtypeStruct((NPAD, D), jnp.float32),
        ],
    )(p0, p1, dis, prev)


def kernel(edge_index, emb_weight):
    row = edge_index[0]
    col = edge_index[1]
    emb_p = jnp.zeros((NPAD, D), jnp.float32).at[:N].set(emb_weight)
    ones_ch = jnp.ones((CH,), jnp.float32)
    zeros1d = jnp.zeros((ROWS_T,), jnp.float32)
    zeros2d = jnp.zeros((ZROWS, D), jnp.float32)

    d0, d1 = _deg_sc(col, ones_ch, zeros1d)
    dis, y, out = _tc_prep(d0, d1, emb_p)
    for k in range(1, NLAYERS + 1):
        p0, p1 = _layer_sc(y, row, col, zeros2d)
        y, out = _tc_comb(p0, p1, dis, out, ALPHA[k])
    return _score_sc(out, row, col)
